# baseline (speedup 1.0000x reference)
import functools

import jax
import jax.numpy as jnp
from jax import lax
from jax.experimental import pallas as pl
from jax.experimental.pallas import tpu as pltpu
from jax.experimental.pallas import tpu_sc as plsc

_NC = 2
_NS = 16
_L = 16
_NW = _NC * _NS

_BLK = 50
_NBUF = 4
_LOOK = 2
_CHUNK = 40
_SZ = 40


def _transform(x, w_aug, src2d, et2d, block_n):
    n, d_in = x.shape
    k, _, d_out = w_aug.shape
    nb = n // block_n
    _, ebb, ec = src2d.shape

    def body(x_ref, w_ref, s_ref, t_ref, o_ref, g_ref):
        o_ref[0] = jnp.dot(x_ref[...], w_ref[0],
                           preferred_element_type=jnp.float32)
        g_ref[0] = t_ref[0] * n + s_ref[0]

    return pl.pallas_call(
        body,
        grid=(nb, k),
        in_specs=[
            pl.BlockSpec((block_n, d_in), lambda i, j: (i, 0)),
            pl.BlockSpec((1, d_in, d_out), lambda i, j: (j, 0, 0)),
            pl.BlockSpec((1, ebb, ec), lambda i, j: (i, 0, 0)),
            pl.BlockSpec((1, ebb, ec), lambda i, j: (i, 0, 0)),
        ],
        out_specs=[
            pl.BlockSpec((1, block_n, d_out), lambda i, j: (j, i, 0)),
            pl.BlockSpec((1, ebb, ec), lambda i, j: (i, 0, 0)),
        ],
        out_shape=[
            jax.ShapeDtypeStruct((k, n, d_out), jnp.float32),
            jax.ShapeDtypeStruct((nb, ebb, ec), jnp.int32),
        ],
    )(x, w_aug, src2d, et2d)


def _combine_transform(agg, deg, haug_prev, b, w_aug, block_n):
    _, n, d_in = agg.shape
    k, _, d_out = w_aug.shape
    nb = n // block_n
    kprev = haug_prev.shape[0]

    def body(agg_ref, deg_ref, self_ref, b_ref, w_ref, o_ref):
        d = deg_ref[0] + deg_ref[1]
        dc = jnp.maximum(d[:, 0:1], 1.0)
        a = (agg_ref[0] + agg_ref[1]) / dc
        h = jnp.maximum(a + self_ref[0] + b_ref[...], 0.0)
        o_ref[0] = jnp.dot(h, w_ref[0], preferred_element_type=jnp.float32)

    return pl.pallas_call(
        body,
        grid=(nb, k),
        in_specs=[
            pl.BlockSpec((2, block_n, d_in), lambda i, j: (0, i, 0)),
            pl.BlockSpec((2, block_n, _L), lambda i, j: (0, i, 0)),
            pl.BlockSpec((1, block_n, d_in), lambda i, j: (kprev - 1, i, 0)),
            pl.BlockSpec((1, d_in), lambda i, j: (0, 0)),
            pl.BlockSpec((1, d_in, d_out), lambda i, j: (j, 0, 0)),
        ],
        out_specs=pl.BlockSpec((1, block_n, d_out), lambda i, j: (j, i, 0)),
        out_shape=jax.ShapeDtypeStruct((k, n, d_out), jnp.float32),
    )(agg, deg, haug_prev, b.reshape(1, d_in), w_aug)


def _finalize(agg, deg, haug_prev, b, block_n):
    _, n, d_out = agg.shape
    nb = n // block_n
    kprev = haug_prev.shape[0]

    def body(agg_ref, deg_ref, self_ref, b_ref, o_ref):
        d = deg_ref[0] + deg_ref[1]
        dc = jnp.maximum(d[:, 0:1], 1.0)
        o_ref[...] = (agg_ref[0] + agg_ref[1]) / dc + self_ref[0] + b_ref[...]

    return pl.pallas_call(
        body,
        grid=(nb,),
        in_specs=[
            pl.BlockSpec((2, block_n, d_out), lambda i: (0, i, 0)),
            pl.BlockSpec((2, block_n, _L), lambda i: (0, i, 0)),
            pl.BlockSpec((1, block_n, d_out), lambda i: (kprev - 1, i, 0)),
            pl.BlockSpec((1, d_out), lambda i: (0, 0)),
        ],
        out_specs=pl.BlockSpec((block_n, d_out), lambda i: (i, 0)),
        out_shape=jax.ShapeDtypeStruct((n, d_out), jnp.float32),
    )(agg, deg, haug_prev, b.reshape(1, d_out))


def _sc_aggregate(hall, gidx_blk, dst_blk, zrow, ones, n_nodes,
                  with_deg):
    d_dim = hall.shape[1]
    eb = gidx_blk.shape[0]
    nblk = eb // _NW
    nch = nblk // _CHUNK
    nsl = n_nodes // _SZ
    spt = (nsl + _NS - 1) // _NS
    assert nblk % _CHUNK == 0 and nblk % 8 == 0 and _CHUNK >= _NBUF

    mesh = plsc.VectorSubcoreMesh(core_axis_name="c", subcore_axis_name="s")

    out_type = [jax.ShapeDtypeStruct((2 * n_nodes, d_dim), jnp.float32)]
    scratch = [pltpu.VMEM_SHARED((n_nodes, d_dim), jnp.float32)]
    if with_deg:
        out_type.append(jax.ShapeDtypeStruct((2 * n_nodes, _L), jnp.float32))
        scratch.append(pltpu.VMEM_SHARED((n_nodes, _L), jnp.float32))
    scratch += [
        pltpu.VMEM((2, _CHUNK, _BLK), jnp.int32),
        pltpu.VMEM((2, _CHUNK, _BLK), jnp.int32),
        pltpu.VMEM((_NBUF, _BLK, d_dim), jnp.float32),
        pltpu.VMEM((_SZ, d_dim), jnp.float32),
    ]
    if with_deg:
        scratch += [pltpu.VMEM((_BLK, _L), jnp.float32)]
    nsem = _NBUF * (3 if with_deg else 2) + 4 + 2 * _NBUF
    scratch += [pltpu.SemaphoreType.DMA] * nsem

    @functools.partial(
        pl.kernel,
        out_type=tuple(out_type),
        mesh=mesh,
        compiler_params=pltpu.CompilerParams(use_tc_tiling_on_sc=False),
        scratch_types=scratch,
    )
    def k(*args):
        it = iter(args)
        hall_ref, gidx_ref, dst_ref, zrow_ref = (
            next(it), next(it), next(it), next(it))
        ones_ref = next(it) if with_deg else None
        oagg_ref = next(it)
        odeg_ref = next(it) if with_deg else None
        agg_sp = next(it)
        deg_sp = next(it) if with_deg else None
        gidx_v, dst_v, rows_v, zrow_v = next(it), next(it), next(it), next(it)
        ones_v = next(it) if with_deg else None
        zdeg_v = zrow_v.at[:, pl.ds(0, _L)] if with_deg else None
        rest = list(it)
        gsem = rest[:_NBUF]
        ssem = rest[_NBUF:2 * _NBUF]
        dsem = rest[2 * _NBUF:3 * _NBUF] if with_deg else []
        tail = rest[3 * _NBUF:] if with_deg else rest[2 * _NBUF:]
        psem = tail[0:2]
        zsem = tail[2:4]
        wisem = tail[4:4 + _NBUF]
        wosem = tail[4 + _NBUF:4 + 2 * _NBUF]

        cid = lax.axis_index("c")
        sid = lax.axis_index("s")
        wid = sid * _NC + cid
        bbase = wid * nblk

        pltpu.sync_copy(zrow_ref, zrow_v)
        if with_deg:
            pltpu.sync_copy(ones_ref, ones_v)

        zd = []
        for t in range(spt - 1):
            s = t * _NS + sid
            zd.append(pltpu.async_copy(
                zrow_v, agg_sp.at[pl.ds(s * _SZ, _SZ)], zsem[0]))
            if with_deg:
                zd.append(pltpu.async_copy(
                    zdeg_v, deg_sp.at[pl.ds(s * _SZ, _SZ)], zsem[1]))
        st = (spt - 1) * _NS + sid

        @pl.when(st < nsl)
        def _():
            pltpu.sync_copy(zrow_v, agg_sp.at[pl.ds(st * _SZ, _SZ)])
            if with_deg:
                pltpu.sync_copy(zdeg_v, deg_sp.at[pl.ds(st * _SZ, _SZ)])

        for d in zd:
            d.wait()

        def prefetch(ci, b):
            pltpu.async_copy(
                gidx_ref.at[pl.ds(bbase + ci * _CHUNK, _CHUNK)],
                gidx_v.at[b], psem[0])
            pltpu.async_copy(
                dst_ref.at[pl.ds(bbase + ci * _CHUNK, _CHUNK)],
                dst_v.at[b], psem[1])

        def pwait(b):
            pltpu.make_async_copy(
                gidx_ref.at[pl.ds(0, _CHUNK)], gidx_v.at[b], psem[0]).wait()
            pltpu.make_async_copy(
                dst_ref.at[pl.ds(0, _CHUNK)], dst_v.at[b], psem[1]).wait()

        prefetch(0, 0)
        plsc.subcore_barrier()

        def chunk(ci, carry):
            b = ci % 2
            pwait(b)

            @pl.when(ci + 1 < nch)
            def _():
                prefetch(ci + 1, 1 - b)

            def gather(blk, buf):
                return pltpu.async_copy(
                    hall_ref.at[gidx_v.at[b, blk]], rows_v.at[buf],
                    gsem[buf])

            gd = {i: gather(i, i % _NBUF) for i in range(_LOOK)}
            sd = {}
            dd = {}
            for i in range(_CHUNK):
                buf = i % _NBUF
                gd[i].wait()
                sd[i] = pltpu.async_copy(
                    rows_v.at[buf], agg_sp.at[dst_v.at[b, i]], ssem[buf],
                    add=True)
                if with_deg:
                    dd[i] = pltpu.async_copy(
                        ones_v, deg_sp.at[dst_v.at[b, i]], dsem[buf],
                        add=True)
                nxt = i + _LOOK
                if nxt < _CHUNK:
                    prev = nxt - _NBUF
                    if prev >= 0:
                        sd[prev].wait()
                        if with_deg:
                            dd[prev].wait()
                    gd[nxt] = gather(nxt, nxt % _NBUF)
            waited = {n - _NBUF for n in range(_LOOK, _CHUNK) if n - _NBUF >= 0}
            for i in range(_CHUNK):
                if i not in waited:
                    sd[i].wait()
                    if with_deg:
                        dd[i].wait()
            return carry

        lax.fori_loop(0, nch, chunk, 0)

        plsc.subcore_barrier()
        wout = {}
        for t in range(spt - 1):
            r0 = (t * _NS + sid) * _SZ
            buf = t % _NBUF
            stg = rows_v.at[buf, pl.ds(0, _SZ)]
            if t >= _NBUF:
                wout[buf].wait()
            pltpu.async_copy(agg_sp.at[pl.ds(r0, _SZ)], stg,
                             wisem[buf]).wait()
            wout[buf] = pltpu.async_copy(
                stg, oagg_ref.at[pl.ds(cid * n_nodes + r0, _SZ)], wosem[buf])
        for buf in wout:
            wout[buf].wait()
        rt = (spt - 1) * _NS + sid

        @pl.when(rt < nsl)
        def _():
            r0 = rt * _SZ
            pltpu.sync_copy(agg_sp.at[pl.ds(r0, _SZ)], zrow_v)
            pltpu.sync_copy(
                zrow_v, oagg_ref.at[pl.ds(cid * n_nodes + r0, _SZ)])

        if with_deg:
            for t in range(spt):
                s = t * _NS + sid

                @pl.when(s < nsl)
                def _():
                    r0 = s * _SZ
                    pltpu.sync_copy(deg_sp.at[pl.ds(r0, _SZ)], zdeg_v)
                    pltpu.sync_copy(
                        zdeg_v, odeg_ref.at[pl.ds(cid * n_nodes + r0, _SZ)])

    if with_deg:
        return k(hall, gidx_blk, dst_blk, zrow, ones)
    return k(hall, gidx_blk, dst_blk, zrow)


def kernel(features, edge_index, edge_types, W1, Wself1, b1, W2, Wself2, b2):
    n, d_in = features.shape
    r = W1.shape[0]
    d_hid = W1.shape[2]
    d_out = W2.shape[2]
    e = edge_types.shape[0]
    kb = r + 1
    block_n = 2000

    w1_aug = jnp.concatenate([W1, Wself1[None]], axis=0)
    w2_aug = jnp.concatenate([W2, Wself2[None]], axis=0)

    nb = n // block_n
    src2d = edge_index[0].reshape(nb, e // (128 * nb), 128)
    et2d = edge_types.reshape(nb, e // (128 * nb), 128)
    dst_blk = edge_index[1].reshape(e // _BLK, _BLK)

    zrow = jnp.zeros((_SZ, d_hid), jnp.float32)
    ones = jnp.ones((_BLK, _L), jnp.float32)

    haug1, gidx2d = _transform(features, w1_aug, src2d, et2d, block_n)
    gidx_blk = gidx2d.reshape(e // _BLK, _BLK)
    agg1f, deg1f = _sc_aggregate(
        haug1.reshape(kb * n, d_hid), gidx_blk, dst_blk, zrow, ones, n,
        with_deg=True)
    agg1 = agg1f.reshape(2, n, d_hid)
    deg = deg1f.reshape(2, n, _L)

    haug2 = _combine_transform(agg1, deg, haug1, b1, w2_aug, block_n)
    (agg2f,) = _sc_aggregate(
        haug2.reshape(kb * n, d_out), gidx_blk, dst_blk, zrow, ones, n,
        with_deg=False)
    agg2 = agg2f.reshape(2, n, d_out)

    return _finalize(agg2, deg, haug2, b2, block_n)


# speedup vs baseline: 1.0448x; 1.0448x over previous
import functools

import jax
import jax.numpy as jnp
from jax import lax
from jax.experimental import pallas as pl
from jax.experimental.pallas import tpu as pltpu
from jax.experimental.pallas import tpu_sc as plsc

_NC = 2
_NS = 16
_L = 16
_NW = _NC * _NS

_BLK = 50
_NBUF = 5
_LOOK = 3
_CHUNK = 10
_SZ = 40


def _transform(x, w_aug, src2d, et2d, block_n):
    n, d_in = x.shape
    k, _, d_out = w_aug.shape
    nb = n // block_n
    _, ebb, ec = src2d.shape

    def body(x_ref, w_ref, s_ref, t_ref, o_ref, g_ref):
        o_ref[0] = jnp.dot(x_ref[...], w_ref[0],
                           preferred_element_type=jnp.float32)
        g_ref[0] = t_ref[0] * n + s_ref[0]

    return pl.pallas_call(
        body,
        grid=(nb, k),
        in_specs=[
            pl.BlockSpec((block_n, d_in), lambda i, j: (i, 0)),
            pl.BlockSpec((1, d_in, d_out), lambda i, j: (j, 0, 0)),
            pl.BlockSpec((1, ebb, ec), lambda i, j: (i, 0, 0)),
            pl.BlockSpec((1, ebb, ec), lambda i, j: (i, 0, 0)),
        ],
        out_specs=[
            pl.BlockSpec((1, block_n, d_out), lambda i, j: (j, i, 0)),
            pl.BlockSpec((1, ebb, ec), lambda i, j: (i, 0, 0)),
        ],
        out_shape=[
            jax.ShapeDtypeStruct((k, n, d_out), jnp.float32),
            jax.ShapeDtypeStruct((nb, ebb, ec), jnp.int32),
        ],
    )(x, w_aug, src2d, et2d)


def _combine_transform(agg, deg, haug_prev, b, w_aug, block_n):
    _, n, d_in = agg.shape
    k, _, d_out = w_aug.shape
    nb = n // block_n
    kprev = haug_prev.shape[0]

    def body(agg_ref, deg_ref, self_ref, b_ref, w_ref, o_ref):
        d = deg_ref[0] + deg_ref[1]
        dc = jnp.maximum(d[:, 0:1], 1.0)
        a = (agg_ref[0] + agg_ref[1]) / dc
        h = jnp.maximum(a + self_ref[0] + b_ref[...], 0.0)
        o_ref[0] = jnp.dot(h, w_ref[0], preferred_element_type=jnp.float32)

    return pl.pallas_call(
        body,
        grid=(nb, k),
        in_specs=[
            pl.BlockSpec((2, block_n, d_in), lambda i, j: (0, i, 0)),
            pl.BlockSpec((2, block_n, _L), lambda i, j: (0, i, 0)),
            pl.BlockSpec((1, block_n, d_in), lambda i, j: (kprev - 1, i, 0)),
            pl.BlockSpec((1, d_in), lambda i, j: (0, 0)),
            pl.BlockSpec((1, d_in, d_out), lambda i, j: (j, 0, 0)),
        ],
        out_specs=pl.BlockSpec((1, block_n, d_out), lambda i, j: (j, i, 0)),
        out_shape=jax.ShapeDtypeStruct((k, n, d_out), jnp.float32),
    )(agg, deg, haug_prev, b.reshape(1, d_in), w_aug)


def _finalize(agg, deg, haug_prev, b, block_n):
    _, n, d_out = agg.shape
    nb = n // block_n
    kprev = haug_prev.shape[0]

    def body(agg_ref, deg_ref, self_ref, b_ref, o_ref):
        d = deg_ref[0] + deg_ref[1]
        dc = jnp.maximum(d[:, 0:1], 1.0)
        o_ref[...] = (agg_ref[0] + agg_ref[1]) / dc + self_ref[0] + b_ref[...]

    return pl.pallas_call(
        body,
        grid=(nb,),
        in_specs=[
            pl.BlockSpec((2, block_n, d_out), lambda i: (0, i, 0)),
            pl.BlockSpec((2, block_n, _L), lambda i: (0, i, 0)),
            pl.BlockSpec((1, block_n, d_out), lambda i: (kprev - 1, i, 0)),
            pl.BlockSpec((1, d_out), lambda i: (0, 0)),
        ],
        out_specs=pl.BlockSpec((block_n, d_out), lambda i: (i, 0)),
        out_shape=jax.ShapeDtypeStruct((n, d_out), jnp.float32),
    )(agg, deg, haug_prev, b.reshape(1, d_out))


def _sc_aggregate(hall, gidx_blk, dst_blk, zrow, ones, n_nodes,
                  with_deg):
    d_dim = hall.shape[1]
    eb = gidx_blk.shape[0]
    nblk = eb // _NW
    nch = nblk // _CHUNK
    nsl = n_nodes // _SZ
    spt = (nsl + _NS - 1) // _NS
    assert nblk % _CHUNK == 0 and nblk % 8 == 0 and _CHUNK >= _NBUF

    mesh = plsc.VectorSubcoreMesh(core_axis_name="c", subcore_axis_name="s")

    out_type = [jax.ShapeDtypeStruct((2 * n_nodes, d_dim), jnp.float32)]
    scratch = [pltpu.VMEM_SHARED((n_nodes, d_dim), jnp.float32)]
    if with_deg:
        out_type.append(jax.ShapeDtypeStruct((2 * n_nodes, _L), jnp.float32))
        scratch.append(pltpu.VMEM_SHARED((n_nodes, _L), jnp.float32))
    scratch += [
        pltpu.VMEM((2, _CHUNK, _BLK), jnp.int32),
        pltpu.VMEM((2, _CHUNK, _BLK), jnp.int32),
        pltpu.VMEM((_NBUF, _BLK, d_dim), jnp.float32),
        pltpu.VMEM((_SZ, d_dim), jnp.float32),
    ]
    if with_deg:
        scratch += [pltpu.VMEM((_BLK, _L), jnp.float32)]
    nsem = _NBUF * (3 if with_deg else 2) + 4 + 2 * _NBUF
    scratch += [pltpu.SemaphoreType.DMA] * nsem

    @functools.partial(
        pl.kernel,
        out_type=tuple(out_type),
        mesh=mesh,
        compiler_params=pltpu.CompilerParams(use_tc_tiling_on_sc=False),
        scratch_types=scratch,
    )
    def k(*args):
        it = iter(args)
        hall_ref, gidx_ref, dst_ref, zrow_ref = (
            next(it), next(it), next(it), next(it))
        ones_ref = next(it) if with_deg else None
        oagg_ref = next(it)
        odeg_ref = next(it) if with_deg else None
        agg_sp = next(it)
        deg_sp = next(it) if with_deg else None
        gidx_v, dst_v, rows_v, zrow_v = next(it), next(it), next(it), next(it)
        ones_v = next(it) if with_deg else None
        zdeg_v = zrow_v.at[:, pl.ds(0, _L)] if with_deg else None
        rest = list(it)
        gsem = rest[:_NBUF]
        ssem = rest[_NBUF:2 * _NBUF]
        dsem = rest[2 * _NBUF:3 * _NBUF] if with_deg else []
        tail = rest[3 * _NBUF:] if with_deg else rest[2 * _NBUF:]
        psem = tail[0:2]
        zsem = tail[2:4]
        wisem = tail[4:4 + _NBUF]
        wosem = tail[4 + _NBUF:4 + 2 * _NBUF]

        cid = lax.axis_index("c")
        sid = lax.axis_index("s")
        wid = sid * _NC + cid
        bbase = wid * nblk

        pltpu.sync_copy(zrow_ref, zrow_v)
        if with_deg:
            pltpu.sync_copy(ones_ref, ones_v)

        zd = []
        for t in range(spt - 1):
            s = t * _NS + sid
            zd.append(pltpu.async_copy(
                zrow_v, agg_sp.at[pl.ds(s * _SZ, _SZ)], zsem[0]))
            if with_deg:
                zd.append(pltpu.async_copy(
                    zdeg_v, deg_sp.at[pl.ds(s * _SZ, _SZ)], zsem[1]))
        st = (spt - 1) * _NS + sid

        @pl.when(st < nsl)
        def _():
            pltpu.sync_copy(zrow_v, agg_sp.at[pl.ds(st * _SZ, _SZ)])
            if with_deg:
                pltpu.sync_copy(zdeg_v, deg_sp.at[pl.ds(st * _SZ, _SZ)])

        for d in zd:
            d.wait()

        def prefetch(ci, b):
            pltpu.async_copy(
                gidx_ref.at[pl.ds(bbase + ci * _CHUNK, _CHUNK)],
                gidx_v.at[b], psem[0])
            pltpu.async_copy(
                dst_ref.at[pl.ds(bbase + ci * _CHUNK, _CHUNK)],
                dst_v.at[b], psem[1])

        def pwait(b):
            pltpu.make_async_copy(
                gidx_ref.at[pl.ds(0, _CHUNK)], gidx_v.at[b], psem[0]).wait()
            pltpu.make_async_copy(
                dst_ref.at[pl.ds(0, _CHUNK)], dst_v.at[b], psem[1]).wait()

        prefetch(0, 0)
        plsc.subcore_barrier()

        def chunk(ci, carry):
            b = ci % 2
            pwait(b)

            @pl.when(ci + 1 < nch)
            def _():
                prefetch(ci + 1, 1 - b)

            def gather(blk, buf):
                return pltpu.async_copy(
                    hall_ref.at[gidx_v.at[b, blk]], rows_v.at[buf],
                    gsem[buf])

            gd = {i: gather(i, i % _NBUF) for i in range(_LOOK)}
            sd = {}
            dd = {}
            for i in range(_CHUNK):
                buf = i % _NBUF
                gd[i].wait()
                sd[i] = pltpu.async_copy(
                    rows_v.at[buf], agg_sp.at[dst_v.at[b, i]], ssem[buf],
                    add=True)
                if with_deg:
                    dd[i] = pltpu.async_copy(
                        ones_v, deg_sp.at[dst_v.at[b, i]], dsem[buf],
                        add=True)
                nxt = i + _LOOK
                if nxt < _CHUNK:
                    prev = nxt - _NBUF
                    if prev >= 0:
                        sd[prev].wait()
                        if with_deg:
                            dd[prev].wait()
                    gd[nxt] = gather(nxt, nxt % _NBUF)
            waited = {n - _NBUF for n in range(_LOOK, _CHUNK) if n - _NBUF >= 0}
            for i in range(_CHUNK):
                if i not in waited:
                    sd[i].wait()
                    if with_deg:
                        dd[i].wait()
            return carry

        lax.fori_loop(0, nch, chunk, 0)

        plsc.subcore_barrier()
        wout = {}
        for t in range(spt - 1):
            r0 = (t * _NS + sid) * _SZ
            buf = t % _NBUF
            stg = rows_v.at[buf, pl.ds(0, _SZ)]
            if t >= _NBUF:
                wout[buf].wait()
            pltpu.async_copy(agg_sp.at[pl.ds(r0, _SZ)], stg,
                             wisem[buf]).wait()
            wout[buf] = pltpu.async_copy(
                stg, oagg_ref.at[pl.ds(cid * n_nodes + r0, _SZ)], wosem[buf])
        for buf in wout:
            wout[buf].wait()
        rt = (spt - 1) * _NS + sid

        @pl.when(rt < nsl)
        def _():
            r0 = rt * _SZ
            pltpu.sync_copy(agg_sp.at[pl.ds(r0, _SZ)], zrow_v)
            pltpu.sync_copy(
                zrow_v, oagg_ref.at[pl.ds(cid * n_nodes + r0, _SZ)])

        if with_deg:
            for t in range(spt):
                s = t * _NS + sid

                @pl.when(s < nsl)
                def _():
                    r0 = s * _SZ
                    pltpu.sync_copy(deg_sp.at[pl.ds(r0, _SZ)], zdeg_v)
                    pltpu.sync_copy(
                        zdeg_v, odeg_ref.at[pl.ds(cid * n_nodes + r0, _SZ)])

    if with_deg:
        return k(hall, gidx_blk, dst_blk, zrow, ones)
    return k(hall, gidx_blk, dst_blk, zrow)


def kernel(features, edge_index, edge_types, W1, Wself1, b1, W2, Wself2, b2):
    n, d_in = features.shape
    r = W1.shape[0]
    d_hid = W1.shape[2]
    d_out = W2.shape[2]
    e = edge_types.shape[0]
    kb = r + 1
    block_n = 2000

    w1_aug = jnp.concatenate([W1, Wself1[None]], axis=0)
    w2_aug = jnp.concatenate([W2, Wself2[None]], axis=0)

    nb = n // block_n
    src2d = edge_index[0].reshape(nb, e // (128 * nb), 128)
    et2d = edge_types.reshape(nb, e // (128 * nb), 128)
    dst_blk = edge_index[1].reshape(e // _BLK, _BLK)

    zrow = jnp.zeros((_SZ, d_hid), jnp.float32)
    ones = jnp.ones((_BLK, _L), jnp.float32)

    haug1, gidx2d = _transform(features, w1_aug, src2d, et2d, block_n)
    gidx_blk = gidx2d.reshape(e // _BLK, _BLK)
    agg1f, deg1f = _sc_aggregate(
        haug1.reshape(kb * n, d_hid), gidx_blk, dst_blk, zrow, ones, n,
        with_deg=True)
    agg1 = agg1f.reshape(2, n, d_hid)
    deg = deg1f.reshape(2, n, _L)

    haug2 = _combine_transform(agg1, deg, haug1, b1, w2_aug, block_n)
    (agg2f,) = _sc_aggregate(
        haug2.reshape(kb * n, d_out), gidx_blk, dst_blk, zrow, ones, n,
        with_deg=False)
    agg2 = agg2f.reshape(2, n, d_out)

    return _finalize(agg2, deg, haug2, b2, block_n)


# speedup vs baseline: 1.1238x; 1.0756x over previous
import functools

import jax
import jax.numpy as jnp
from jax import lax
from jax.experimental import pallas as pl
from jax.experimental.pallas import tpu as pltpu
from jax.experimental.pallas import tpu_sc as plsc

_NC = 2
_NS = 16
_L = 16
_NW = _NC * _NS

_BLK = 50
_NBUF = 4
_LOOK = 3
_CHUNK = 40
_SZ = 40


def _transform(x, w_aug, src2d, et2d, block_n):
    n, d_in = x.shape
    k, _, d_out = w_aug.shape
    nb = n // block_n
    _, ebb, ec = src2d.shape

    def body(x_ref, w_ref, s_ref, t_ref, o_ref, g_ref):
        o_ref[0] = jnp.dot(x_ref[...], w_ref[0],
                           preferred_element_type=jnp.float32)
        g_ref[0] = t_ref[0] * n + s_ref[0]

    return pl.pallas_call(
        body,
        grid=(nb, k),
        in_specs=[
            pl.BlockSpec((block_n, d_in), lambda i, j: (i, 0)),
            pl.BlockSpec((1, d_in, d_out), lambda i, j: (j, 0, 0)),
            pl.BlockSpec((1, ebb, ec), lambda i, j: (i, 0, 0)),
            pl.BlockSpec((1, ebb, ec), lambda i, j: (i, 0, 0)),
        ],
        out_specs=[
            pl.BlockSpec((1, block_n, d_out), lambda i, j: (j, i, 0)),
            pl.BlockSpec((1, ebb, ec), lambda i, j: (i, 0, 0)),
        ],
        out_shape=[
            jax.ShapeDtypeStruct((k, n, d_out), jnp.float32),
            jax.ShapeDtypeStruct((nb, ebb, ec), jnp.int32),
        ],
    )(x, w_aug, src2d, et2d)


def _combine_transform(agg, deg, haug_prev, b, w_aug, block_n):
    _, n, d_in = agg.shape
    k, _, d_out = w_aug.shape
    nb = n // block_n
    kprev = haug_prev.shape[0]

    def body(agg_ref, deg_ref, self_ref, b_ref, w_ref, o_ref):
        d = deg_ref[0] + deg_ref[1]
        dc = jnp.maximum(d[:, 0:1], 1.0)
        a = (agg_ref[0] + agg_ref[1]) / dc
        h = jnp.maximum(a + self_ref[0] + b_ref[...], 0.0)
        o_ref[0] = jnp.dot(h, w_ref[0], preferred_element_type=jnp.float32)

    return pl.pallas_call(
        body,
        grid=(nb, k),
        in_specs=[
            pl.BlockSpec((2, block_n, d_in), lambda i, j: (0, i, 0)),
            pl.BlockSpec((2, block_n, _L), lambda i, j: (0, i, 0)),
            pl.BlockSpec((1, block_n, d_in), lambda i, j: (kprev - 1, i, 0)),
            pl.BlockSpec((1, d_in), lambda i, j: (0, 0)),
            pl.BlockSpec((1, d_in, d_out), lambda i, j: (j, 0, 0)),
        ],
        out_specs=pl.BlockSpec((1, block_n, d_out), lambda i, j: (j, i, 0)),
        out_shape=jax.ShapeDtypeStruct((k, n, d_out), jnp.float32),
    )(agg, deg, haug_prev, b.reshape(1, d_in), w_aug)


def _finalize(agg, deg, haug_prev, b, block_n):
    _, n, d_out = agg.shape
    nb = n // block_n
    kprev = haug_prev.shape[0]

    def body(agg_ref, deg_ref, self_ref, b_ref, o_ref):
        d = deg_ref[0] + deg_ref[1]
        dc = jnp.maximum(d[:, 0:1], 1.0)
        o_ref[...] = (agg_ref[0] + agg_ref[1]) / dc + self_ref[0] + b_ref[...]

    return pl.pallas_call(
        body,
        grid=(nb,),
        in_specs=[
            pl.BlockSpec((2, block_n, d_out), lambda i: (0, i, 0)),
            pl.BlockSpec((2, block_n, _L), lambda i: (0, i, 0)),
            pl.BlockSpec((1, block_n, d_out), lambda i: (kprev - 1, i, 0)),
            pl.BlockSpec((1, d_out), lambda i: (0, 0)),
        ],
        out_specs=pl.BlockSpec((block_n, d_out), lambda i: (i, 0)),
        out_shape=jax.ShapeDtypeStruct((n, d_out), jnp.float32),
    )(agg, deg, haug_prev, b.reshape(1, d_out))


def _sc_aggregate(hall, gidx_blk, dst_blk, zrow, ones, n_nodes,
                  with_deg):
    d_dim = hall.shape[1]
    eb = gidx_blk.shape[0]
    nblk = eb // _NW
    nch = nblk // _CHUNK
    nsl = n_nodes // _SZ
    spt = (nsl + _NS - 1) // _NS
    assert nblk % _CHUNK == 0 and nblk % 8 == 0 and _CHUNK >= _NBUF

    mesh = plsc.VectorSubcoreMesh(core_axis_name="c", subcore_axis_name="s")

    out_type = [jax.ShapeDtypeStruct((2 * n_nodes, d_dim), jnp.float32)]
    scratch = [pltpu.VMEM_SHARED((n_nodes, d_dim), jnp.float32)]
    if with_deg:
        out_type.append(jax.ShapeDtypeStruct((2 * n_nodes, _L), jnp.float32))
        scratch.append(pltpu.VMEM_SHARED((n_nodes, _L), jnp.float32))
    scratch += [
        pltpu.VMEM((2, _CHUNK, _BLK), jnp.int32),
        pltpu.VMEM((2, _CHUNK, _BLK), jnp.int32),
        pltpu.VMEM((_NBUF, _BLK, d_dim), jnp.float32),
        pltpu.VMEM((_SZ, d_dim), jnp.float32),
    ]
    if with_deg:
        scratch += [pltpu.VMEM((_BLK, _L), jnp.float32)]
    nsem = _NBUF * (3 if with_deg else 2) + 4 + 2 * _NBUF
    scratch += [pltpu.SemaphoreType.DMA] * nsem

    @functools.partial(
        pl.kernel,
        out_type=tuple(out_type),
        mesh=mesh,
        compiler_params=pltpu.CompilerParams(use_tc_tiling_on_sc=False),
        scratch_types=scratch,
    )
    def k(*args):
        it = iter(args)
        hall_ref, gidx_ref, dst_ref, zrow_ref = (
            next(it), next(it), next(it), next(it))
        ones_ref = next(it) if with_deg else None
        oagg_ref = next(it)
        odeg_ref = next(it) if with_deg else None
        agg_sp = next(it)
        deg_sp = next(it) if with_deg else None
        gidx_v, dst_v, rows_v, zrow_v = next(it), next(it), next(it), next(it)
        ones_v = next(it) if with_deg else None
        zdeg_v = zrow_v.at[:, pl.ds(0, _L)] if with_deg else None
        rest = list(it)
        gsem = rest[:_NBUF]
        ssem = rest[_NBUF:2 * _NBUF]
        dsem = rest[2 * _NBUF:3 * _NBUF] if with_deg else []
        tail = rest[3 * _NBUF:] if with_deg else rest[2 * _NBUF:]
        psem = tail[0:2]
        zsem = tail[2:4]
        wisem = tail[4:4 + _NBUF]
        wosem = tail[4 + _NBUF:4 + 2 * _NBUF]

        cid = lax.axis_index("c")
        sid = lax.axis_index("s")
        wid = sid * _NC + cid
        bbase = wid * nblk

        pltpu.sync_copy(zrow_ref, zrow_v)
        if with_deg:
            pltpu.sync_copy(ones_ref, ones_v)

        zd = []
        for t in range(spt - 1):
            s = t * _NS + sid
            zd.append(pltpu.async_copy(
                zrow_v, agg_sp.at[pl.ds(s * _SZ, _SZ)], zsem[0]))
            if with_deg:
                zd.append(pltpu.async_copy(
                    zdeg_v, deg_sp.at[pl.ds(s * _SZ, _SZ)], zsem[1]))
        st = (spt - 1) * _NS + sid

        @pl.when(st < nsl)
        def _():
            pltpu.sync_copy(zrow_v, agg_sp.at[pl.ds(st * _SZ, _SZ)])
            if with_deg:
                pltpu.sync_copy(zdeg_v, deg_sp.at[pl.ds(st * _SZ, _SZ)])

        for d in zd:
            d.wait()

        def prefetch(ci, b):
            pltpu.async_copy(
                gidx_ref.at[pl.ds(bbase + ci * _CHUNK, _CHUNK)],
                gidx_v.at[b], psem[0])
            pltpu.async_copy(
                dst_ref.at[pl.ds(bbase + ci * _CHUNK, _CHUNK)],
                dst_v.at[b], psem[1])

        def pwait(b):
            pltpu.make_async_copy(
                gidx_ref.at[pl.ds(0, _CHUNK)], gidx_v.at[b], psem[0]).wait()
            pltpu.make_async_copy(
                dst_ref.at[pl.ds(0, _CHUNK)], dst_v.at[b], psem[1]).wait()

        prefetch(0, 0)
        plsc.subcore_barrier()

        def chunk(ci, carry):
            b = ci % 2
            pwait(b)

            @pl.when(ci + 1 < nch)
            def _():
                prefetch(ci + 1, 1 - b)

            def gather(blk, buf):
                return pltpu.async_copy(
                    hall_ref.at[gidx_v.at[b, blk]], rows_v.at[buf],
                    gsem[buf])

            gd = {i: gather(i, i % _NBUF) for i in range(_LOOK)}
            sd = {}
            dd = {}
            for i in range(_CHUNK):
                buf = i % _NBUF
                gd[i].wait()
                sd[i] = pltpu.async_copy(
                    rows_v.at[buf], agg_sp.at[dst_v.at[b, i]], ssem[buf],
                    add=True)
                if with_deg:
                    dd[i] = pltpu.async_copy(
                        ones_v, deg_sp.at[dst_v.at[b, i]], dsem[buf],
                        add=True)
                nxt = i + _LOOK
                if nxt < _CHUNK:
                    prev = nxt - _NBUF
                    if prev >= 0:
                        sd[prev].wait()
                        if with_deg:
                            dd[prev].wait()
                    gd[nxt] = gather(nxt, nxt % _NBUF)
            waited = {n - _NBUF for n in range(_LOOK, _CHUNK) if n - _NBUF >= 0}
            for i in range(_CHUNK):
                if i not in waited:
                    sd[i].wait()
                    if with_deg:
                        dd[i].wait()
            return carry

        lax.fori_loop(0, nch, chunk, 0)

        plsc.subcore_barrier()
        wout = {}
        for t in range(spt - 1):
            r0 = (t * _NS + sid) * _SZ
            buf = t % _NBUF
            stg = rows_v.at[buf, pl.ds(0, _SZ)]
            if t >= _NBUF:
                wout[buf].wait()
            pltpu.async_copy(agg_sp.at[pl.ds(r0, _SZ)], stg,
                             wisem[buf]).wait()
            wout[buf] = pltpu.async_copy(
                stg, oagg_ref.at[pl.ds(cid * n_nodes + r0, _SZ)], wosem[buf])
        for buf in wout:
            wout[buf].wait()
        rt = (spt - 1) * _NS + sid

        @pl.when(rt < nsl)
        def _():
            r0 = rt * _SZ
            pltpu.sync_copy(agg_sp.at[pl.ds(r0, _SZ)], zrow_v)
            pltpu.sync_copy(
                zrow_v, oagg_ref.at[pl.ds(cid * n_nodes + r0, _SZ)])

        if with_deg:
            for t in range(spt):
                s = t * _NS + sid

                @pl.when(s < nsl)
                def _():
                    r0 = s * _SZ
                    pltpu.sync_copy(deg_sp.at[pl.ds(r0, _SZ)], zdeg_v)
                    pltpu.sync_copy(
                        zdeg_v, odeg_ref.at[pl.ds(cid * n_nodes + r0, _SZ)])

    if with_deg:
        return k(hall, gidx_blk, dst_blk, zrow, ones)
    return k(hall, gidx_blk, dst_blk, zrow)


def kernel(features, edge_index, edge_types, W1, Wself1, b1, W2, Wself2, b2):
    n, d_in = features.shape
    r = W1.shape[0]
    d_hid = W1.shape[2]
    d_out = W2.shape[2]
    e = edge_types.shape[0]
    kb = r + 1
    block_n = 2000

    w1_aug = jnp.concatenate([W1, Wself1[None]], axis=0)
    w2_aug = jnp.concatenate([W2, Wself2[None]], axis=0)

    nb = n // block_n
    src2d = edge_index[0].reshape(nb, e // (128 * nb), 128)
    et2d = edge_types.reshape(nb, e // (128 * nb), 128)
    dst_blk = edge_index[1].reshape(e // _BLK, _BLK)

    zrow = jnp.zeros((_SZ, d_hid), jnp.float32)
    ones = jnp.ones((_BLK, _L), jnp.float32)

    haug1, gidx2d = _transform(features, w1_aug, src2d, et2d, block_n)
    gidx_blk = gidx2d.reshape(e // _BLK, _BLK)
    agg1f, deg1f = _sc_aggregate(
        haug1.reshape(kb * n, d_hid), gidx_blk, dst_blk, zrow, ones, n,
        with_deg=True)
    agg1 = agg1f.reshape(2, n, d_hid)
    deg = deg1f.reshape(2, n, _L)

    haug2 = _combine_transform(agg1, deg, haug1, b1, w2_aug, block_n)
    (agg2f,) = _sc_aggregate(
        haug2.reshape(kb * n, d_out), gidx_blk, dst_blk, zrow, ones, n,
        with_deg=False)
    agg2 = agg2f.reshape(2, n, d_out)

    return _finalize(agg2, deg, haug2, b2, block_n)


# speedup vs baseline: 1.1311x; 1.0065x over previous
import functools

import jax
import jax.numpy as jnp
from jax import lax
from jax.experimental import pallas as pl
from jax.experimental.pallas import tpu as pltpu
from jax.experimental.pallas import tpu_sc as plsc

_NC = 2
_NS = 16
_L = 16
_NW = _NC * _NS

_BLK = 50
_NBUF = 4
_LOOK = 3
_CHUNK = 40
_SZ = 40


def _transform(x, w_aug, src2d, et2d, block_n):
    n, d_in = x.shape
    k, _, d_out = w_aug.shape
    nb = n // block_n
    _, ebb, ec = src2d.shape

    def body(x_ref, w_ref, s_ref, t_ref, o_ref, g_ref):
        o_ref[0] = jnp.dot(x_ref[...], w_ref[0],
                           preferred_element_type=jnp.float32)
        g_ref[0] = t_ref[0] * n + s_ref[0]

    return pl.pallas_call(
        body,
        grid=(nb, k),
        in_specs=[
            pl.BlockSpec((block_n, d_in), lambda i, j: (i, 0)),
            pl.BlockSpec((1, d_in, d_out), lambda i, j: (j, 0, 0)),
            pl.BlockSpec((1, ebb, ec), lambda i, j: (i, 0, 0)),
            pl.BlockSpec((1, ebb, ec), lambda i, j: (i, 0, 0)),
        ],
        out_specs=[
            pl.BlockSpec((1, block_n, d_out), lambda i, j: (j, i, 0)),
            pl.BlockSpec((1, ebb, ec), lambda i, j: (i, 0, 0)),
        ],
        out_shape=[
            jax.ShapeDtypeStruct((k, n, d_out), jnp.float32),
            jax.ShapeDtypeStruct((nb, ebb, ec), jnp.int32),
        ],
    )(x, w_aug, src2d, et2d)


def _combine_transform(agg, deg, haug_prev, b, w_aug, block_n):
    _, n, d_in = agg.shape
    k, _, d_out = w_aug.shape
    nb = n // block_n
    kprev = haug_prev.shape[0]

    def body(agg_ref, deg_ref, self_ref, b_ref, w_ref, o_ref):
        d = deg_ref[0] + deg_ref[1]
        dc = jnp.maximum(d[:, 0:1], 1.0)
        a = (agg_ref[0] + agg_ref[1]) / dc
        h = jnp.maximum(a + self_ref[0] + b_ref[...], 0.0)
        o_ref[0] = jnp.dot(h, w_ref[0], preferred_element_type=jnp.float32)

    return pl.pallas_call(
        body,
        grid=(nb, k),
        in_specs=[
            pl.BlockSpec((2, block_n, d_in), lambda i, j: (0, i, 0)),
            pl.BlockSpec((2, block_n, _L), lambda i, j: (0, i, 0)),
            pl.BlockSpec((1, block_n, d_in), lambda i, j: (kprev - 1, i, 0)),
            pl.BlockSpec((1, d_in), lambda i, j: (0, 0)),
            pl.BlockSpec((1, d_in, d_out), lambda i, j: (j, 0, 0)),
        ],
        out_specs=pl.BlockSpec((1, block_n, d_out), lambda i, j: (j, i, 0)),
        out_shape=jax.ShapeDtypeStruct((k, n, d_out), jnp.float32),
    )(agg, deg, haug_prev, b.reshape(1, d_in), w_aug)


def _finalize(agg, deg, haug_prev, b, block_n):
    _, n, d_out = agg.shape
    nb = n // block_n
    kprev = haug_prev.shape[0]

    def body(agg_ref, deg_ref, self_ref, b_ref, o_ref):
        d = deg_ref[0] + deg_ref[1]
        dc = jnp.maximum(d[:, 0:1], 1.0)
        o_ref[...] = (agg_ref[0] + agg_ref[1]) / dc + self_ref[0] + b_ref[...]

    return pl.pallas_call(
        body,
        grid=(nb,),
        in_specs=[
            pl.BlockSpec((2, block_n, d_out), lambda i: (0, i, 0)),
            pl.BlockSpec((2, block_n, _L), lambda i: (0, i, 0)),
            pl.BlockSpec((1, block_n, d_out), lambda i: (kprev - 1, i, 0)),
            pl.BlockSpec((1, d_out), lambda i: (0, 0)),
        ],
        out_specs=pl.BlockSpec((block_n, d_out), lambda i: (i, 0)),
        out_shape=jax.ShapeDtypeStruct((n, d_out), jnp.float32),
    )(agg, deg, haug_prev, b.reshape(1, d_out))


def _sc_aggregate(hall, gidx_blk, dst_blk, zrow, ones, n_nodes,
                  with_deg):
    d_dim = hall.shape[1]
    eb = gidx_blk.shape[0]
    nblk = eb // _NW
    nch = nblk // _CHUNK
    nsl = n_nodes // _SZ
    spt = (nsl + _NS - 1) // _NS
    assert nblk % _CHUNK == 0 and nblk % 8 == 0 and _CHUNK >= _NBUF

    mesh = plsc.VectorSubcoreMesh(core_axis_name="c", subcore_axis_name="s")

    out_type = [jax.ShapeDtypeStruct((2 * n_nodes, d_dim), jnp.float32)]
    scratch = [pltpu.VMEM_SHARED((n_nodes, d_dim), jnp.float32)]
    if with_deg:
        out_type.append(jax.ShapeDtypeStruct((2 * n_nodes, _L), jnp.float32))
        scratch.append(pltpu.VMEM_SHARED((n_nodes, _L), jnp.float32))
    scratch += [
        pltpu.VMEM((2, _CHUNK, _BLK), jnp.int32),
        pltpu.VMEM((2, _CHUNK, _BLK), jnp.int32),
        pltpu.VMEM((_NBUF, _BLK, d_dim), jnp.float32),
        pltpu.VMEM((_SZ, d_dim), jnp.float32),
    ]
    if with_deg:
        scratch += [pltpu.VMEM((_BLK, _L), jnp.float32)]
    nsem = _NBUF * (3 if with_deg else 2) + 4 + 2 * _NBUF
    scratch += [pltpu.SemaphoreType.DMA] * nsem

    @functools.partial(
        pl.kernel,
        out_type=tuple(out_type),
        mesh=mesh,
        compiler_params=pltpu.CompilerParams(use_tc_tiling_on_sc=False),
        scratch_types=scratch,
    )
    def k(*args):
        it = iter(args)
        hall_ref, gidx_ref, dst_ref, zrow_ref = (
            next(it), next(it), next(it), next(it))
        ones_ref = next(it) if with_deg else None
        oagg_ref = next(it)
        odeg_ref = next(it) if with_deg else None
        agg_sp = next(it)
        deg_sp = next(it) if with_deg else None
        gidx_v, dst_v, rows_v, zrow_v = next(it), next(it), next(it), next(it)
        ones_v = next(it) if with_deg else None
        zdeg_v = zrow_v.at[:, pl.ds(0, _L)] if with_deg else None
        rest = list(it)
        gsem = rest[:_NBUF]
        ssem = rest[_NBUF:2 * _NBUF]
        dsem = rest[2 * _NBUF:3 * _NBUF] if with_deg else []
        tail = rest[3 * _NBUF:] if with_deg else rest[2 * _NBUF:]
        psem = tail[0:2]
        zsem = tail[2:4]
        wisem = tail[4:4 + _NBUF]
        wosem = tail[4 + _NBUF:4 + 2 * _NBUF]

        cid = lax.axis_index("c")
        sid = lax.axis_index("s")
        wid = sid * _NC + cid
        bbase = wid * nblk

        pltpu.sync_copy(zrow_ref, zrow_v)
        if with_deg:
            pltpu.sync_copy(ones_ref, ones_v)

        zd = []
        for t in range(spt - 1):
            s = t * _NS + sid
            zd.append(pltpu.async_copy(
                zrow_v, agg_sp.at[pl.ds(s * _SZ, _SZ)], zsem[0]))
            if with_deg:
                zd.append(pltpu.async_copy(
                    zdeg_v, deg_sp.at[pl.ds(s * _SZ, _SZ)], zsem[1]))
        st = (spt - 1) * _NS + sid

        @pl.when(st < nsl)
        def _():
            pltpu.sync_copy(zrow_v, agg_sp.at[pl.ds(st * _SZ, _SZ)])
            if with_deg:
                pltpu.sync_copy(zdeg_v, deg_sp.at[pl.ds(st * _SZ, _SZ)])

        for d in zd:
            d.wait()

        def prefetch(ci, b):
            pltpu.async_copy(
                gidx_ref.at[pl.ds(bbase + ci * _CHUNK, _CHUNK)],
                gidx_v.at[b], psem[0])
            pltpu.async_copy(
                dst_ref.at[pl.ds(bbase + ci * _CHUNK, _CHUNK)],
                dst_v.at[b], psem[1])

        def pwait(b):
            pltpu.make_async_copy(
                gidx_ref.at[pl.ds(0, _CHUNK)], gidx_v.at[b], psem[0]).wait()
            pltpu.make_async_copy(
                dst_ref.at[pl.ds(0, _CHUNK)], dst_v.at[b], psem[1]).wait()

        prefetch(0, 0)
        plsc.subcore_barrier()

        def drain_tail(bprev):
            for i in range(_CHUNK - _NBUF, _CHUNK):
                buf = i % _NBUF
                pltpu.make_async_copy(
                    rows_v.at[buf], agg_sp.at[dst_v.at[bprev, i]],
                    ssem[buf]).wait()
                if with_deg:
                    pltpu.make_async_copy(
                        ones_v, deg_sp.at[dst_v.at[bprev, i]],
                        dsem[buf]).wait()

        def chunk(ci, carry):
            b = ci % 2
            pwait(b)

            @pl.when(ci > 0)
            def _():
                drain_tail(1 - b)

            @pl.when(ci + 1 < nch)
            def _():
                prefetch(ci + 1, 1 - b)

            def gather(blk, buf):
                return pltpu.async_copy(
                    hall_ref.at[gidx_v.at[b, blk]], rows_v.at[buf],
                    gsem[buf])

            gd = {i: gather(i, i % _NBUF) for i in range(_LOOK)}
            sd = {}
            dd = {}
            for i in range(_CHUNK):
                buf = i % _NBUF
                gd[i].wait()
                sd[i] = pltpu.async_copy(
                    rows_v.at[buf], agg_sp.at[dst_v.at[b, i]], ssem[buf],
                    add=True)
                if with_deg:
                    dd[i] = pltpu.async_copy(
                        ones_v, deg_sp.at[dst_v.at[b, i]], dsem[buf],
                        add=True)
                nxt = i + _LOOK
                if nxt < _CHUNK:
                    prev = nxt - _NBUF
                    if prev >= 0:
                        sd[prev].wait()
                        if with_deg:
                            dd[prev].wait()
                    gd[nxt] = gather(nxt, nxt % _NBUF)
            waited = {n - _NBUF for n in range(_LOOK, _CHUNK) if n - _NBUF >= 0}
            for i in range(_CHUNK - _NBUF):
                if i not in waited:
                    sd[i].wait()
                    if with_deg:
                        dd[i].wait()
            return carry

        lax.fori_loop(0, nch, chunk, 0)
        drain_tail((nch - 1) % 2)

        plsc.subcore_barrier()
        wout = {}
        for t in range(spt - 1):
            r0 = (t * _NS + sid) * _SZ
            buf = t % _NBUF
            stg = rows_v.at[buf, pl.ds(0, _SZ)]
            if t >= _NBUF:
                wout[buf].wait()
            pltpu.async_copy(agg_sp.at[pl.ds(r0, _SZ)], stg,
                             wisem[buf]).wait()
            wout[buf] = pltpu.async_copy(
                stg, oagg_ref.at[pl.ds(cid * n_nodes + r0, _SZ)], wosem[buf])
        for buf in wout:
            wout[buf].wait()
        rt = (spt - 1) * _NS + sid

        @pl.when(rt < nsl)
        def _():
            r0 = rt * _SZ
            pltpu.sync_copy(agg_sp.at[pl.ds(r0, _SZ)], zrow_v)
            pltpu.sync_copy(
                zrow_v, oagg_ref.at[pl.ds(cid * n_nodes + r0, _SZ)])

        if with_deg:
            for t in range(spt):
                s = t * _NS + sid

                @pl.when(s < nsl)
                def _():
                    r0 = s * _SZ
                    pltpu.sync_copy(deg_sp.at[pl.ds(r0, _SZ)], zdeg_v)
                    pltpu.sync_copy(
                        zdeg_v, odeg_ref.at[pl.ds(cid * n_nodes + r0, _SZ)])

    if with_deg:
        return k(hall, gidx_blk, dst_blk, zrow, ones)
    return k(hall, gidx_blk, dst_blk, zrow)


def kernel(features, edge_index, edge_types, W1, Wself1, b1, W2, Wself2, b2):
    n, d_in = features.shape
    r = W1.shape[0]
    d_hid = W1.shape[2]
    d_out = W2.shape[2]
    e = edge_types.shape[0]
    kb = r + 1
    block_n = 2000

    w1_aug = jnp.concatenate([W1, Wself1[None]], axis=0)
    w2_aug = jnp.concatenate([W2, Wself2[None]], axis=0)

    nb = n // block_n
    src2d = edge_index[0].reshape(nb, e // (128 * nb), 128)
    et2d = edge_types.reshape(nb, e // (128 * nb), 128)
    dst_blk = edge_index[1].reshape(e // _BLK, _BLK)

    zrow = jnp.zeros((_SZ, d_hid), jnp.float32)
    ones = jnp.ones((_BLK, _L), jnp.float32)

    haug1, gidx2d = _transform(features, w1_aug, src2d, et2d, block_n)
    gidx_blk = gidx2d.reshape(e // _BLK, _BLK)
    agg1f, deg1f = _sc_aggregate(
        haug1.reshape(kb * n, d_hid), gidx_blk, dst_blk, zrow, ones, n,
        with_deg=True)
    agg1 = agg1f.reshape(2, n, d_hid)
    deg = deg1f.reshape(2, n, _L)

    haug2 = _combine_transform(agg1, deg, haug1, b1, w2_aug, block_n)
    (agg2f,) = _sc_aggregate(
        haug2.reshape(kb * n, d_out), gidx_blk, dst_blk, zrow, ones, n,
        with_deg=False)
    agg2 = agg2f.reshape(2, n, d_out)

    return _finalize(agg2, deg, haug2, b2, block_n)
